# baseline (speedup 1.0000x reference)
import jax
import jax.numpy as jnp
from jax.experimental import pallas as pl
from jax.experimental.pallas import tpu as pltpu

_REG_MAX = 16
_NC = 80
_NREG = 4 * (_REG_MAX + 1)
_STRIDES = (32, 16, 8)
_HWS = ((20, 20), (40, 40), (80, 80))
_OFFSET = 0.5


def _head_body(H, W, feat_ref,
               cfw, cfb, ccw, cbs, cbb,
               rfw, rfb, rcw, rbs, rbb,
               cpw, cpb, rpw, rpb,
               cls_out, reg_out):
    C = feat_ref.shape[1]
    L = H * W
    feat = feat_ref[0].reshape(C, L)
    avg = jnp.sum(feat, axis=1, keepdims=True) * (1.0 / L)
    avg_b = avg.astype(jnp.bfloat16)

    def ese(fw, fb, cw, s, b):
        g = jnp.dot(fw[...], avg_b, preferred_element_type=jnp.float32)
        gate = jax.nn.sigmoid(g + fb[...])
        gated = (feat * gate).astype(jnp.bfloat16)
        x = jnp.dot(cw[...], gated, preferred_element_type=jnp.float32)
        z = x * s[...] + b[...]
        return z * jax.nn.sigmoid(z)

    ese_c = ese(cfw, cfb, ccw, cbs, cbb)
    ese_r = ese(rfw, rfb, rcw, rbs, rbb)

    cx = jax.lax.broadcasted_iota(jnp.int32, (1, L), 1) % W

    def conv3x3(xin, w_ref):
        xb = xin.astype(jnp.bfloat16)
        zrow = jnp.zeros((C, W), jnp.bfloat16)
        slices = (
            jnp.concatenate([zrow, xb[:, :L - W]], axis=1),
            xb,
            jnp.concatenate([xb[:, W:], zrow], axis=1),
        )
        P = []
        for kx in range(3):
            acc = None
            for ky in range(3):
                t = jnp.dot(w_ref[ky * 3 + kx], slices[ky],
                            preferred_element_type=jnp.float32)
                acc = t if acc is None else acc + t
            P.append(acc)
        left = pltpu.roll(P[0], 1, axis=1)
        right = pltpu.roll(P[2], L - 1, axis=1)
        return (P[1] + jnp.where(cx >= 1, left, 0.0)
                + jnp.where(cx <= W - 2, right, 0.0))

    logit = conv3x3(ese_c + feat, cpw) + cpb[...]
    cls_out[0] = jax.nn.sigmoid(logit)

    r = conv3x3(ese_r, rpw) + rpb[...]
    proj = jax.lax.broadcasted_iota(
        jnp.int32, (_REG_MAX + 1, 1), 0).astype(jnp.float32)
    rows = []
    for f in range(4):
        blk = r[17 * f:17 * (f + 1), :]
        m = jnp.max(blk, axis=0, keepdims=True)
        e = jnp.exp(blk - m)
        s = jnp.sum(e, axis=0, keepdims=True)
        wsum = jnp.sum(e * proj, axis=0, keepdims=True)
        rows.append(wsum / s)
    reg_out[0] = jnp.concatenate(rows, axis=0)


def _head_scale(feat, ps, H, W, interpret=False):
    (cfw, cfb, ccw, cbs, cbb, rfw, rfb, rcw, rbs, rbb,
     pcw, pcb, prw, prb) = ps
    B, C, _, _ = feat.shape
    L = H * W
    bf = jnp.bfloat16
    featr = feat
    col = lambda v: v.reshape(-1, 1)
    w11 = lambda w: w.reshape(w.shape[0], w.shape[1]).astype(bf)
    w33 = lambda w: w.transpose(2, 3, 0, 1).reshape(9, w.shape[0], w.shape[1]).astype(bf)

    args = (featr,
            w11(cfw), col(cfb), w11(ccw), col(cbs), col(cbb),
            w11(rfw), col(rfb), w11(rcw), col(rbs), col(rbb),
            w33(pcw), col(pcb), w33(prw), col(prb))

    full = lambda a: pl.BlockSpec(a.shape, lambda b, _n=a.ndim: (0,) * _n)
    in_specs = [pl.BlockSpec((1, C, H, W), lambda b: (b, 0, 0, 0))]
    in_specs += [full(a) for a in args[1:]]

    import functools
    body = functools.partial(_head_body, H, W)
    cls_s, reg_d = pl.pallas_call(
        body,
        grid=(B,),
        in_specs=in_specs,
        out_specs=[pl.BlockSpec((1, _NC, L), lambda b: (b, 0, 0)),
                   pl.BlockSpec((1, 4, L), lambda b: (b, 0, 0))],
        out_shape=[jax.ShapeDtypeStruct((B, _NC, L), jnp.float32),
                   jax.ShapeDtypeStruct((B, 4, L), jnp.float32)],
        compiler_params=pltpu.CompilerParams(
            dimension_semantics=("arbitrary",)),
        interpret=interpret,
    )(*args)
    return cls_s, reg_d


def _anchors():
    pts, st = [], []
    for (h, w), s in zip(_HWS, _STRIDES):
        sx = jnp.arange(w, dtype=jnp.float32) + _OFFSET
        sy = jnp.arange(h, dtype=jnp.float32) + _OFFSET
        yy, xx = jnp.meshgrid(sy, sx, indexing='ij')
        pts.append(jnp.stack([xx, yy], -1).reshape(-1, 2))
        st.append(jnp.full((h * w, 1), s, dtype=jnp.float32))
    return jnp.concatenate(pts, 0), jnp.concatenate(st, 0)


def kernel(feat0, feat1, feat2,
           p0_cls_fc_w, p0_cls_fc_b, p0_cls_conv_w, p0_cls_bn_s, p0_cls_bn_b,
           p0_reg_fc_w, p0_reg_fc_b, p0_reg_conv_w, p0_reg_bn_s, p0_reg_bn_b,
           p0_cls_pred_w, p0_cls_pred_b, p0_reg_pred_w, p0_reg_pred_b,
           p1_cls_fc_w, p1_cls_fc_b, p1_cls_conv_w, p1_cls_bn_s, p1_cls_bn_b,
           p1_reg_fc_w, p1_reg_fc_b, p1_reg_conv_w, p1_reg_bn_s, p1_reg_bn_b,
           p1_cls_pred_w, p1_cls_pred_b, p1_reg_pred_w, p1_reg_pred_b,
           p2_cls_fc_w, p2_cls_fc_b, p2_cls_conv_w, p2_cls_bn_s, p2_cls_bn_b,
           p2_reg_fc_w, p2_reg_fc_b, p2_reg_conv_w, p2_reg_bn_s, p2_reg_bn_b,
           p2_cls_pred_w, p2_cls_pred_b, p2_reg_pred_w, p2_reg_pred_b):
    feats = (feat0, feat1, feat2)
    params = (
        (p0_cls_fc_w, p0_cls_fc_b, p0_cls_conv_w, p0_cls_bn_s, p0_cls_bn_b,
         p0_reg_fc_w, p0_reg_fc_b, p0_reg_conv_w, p0_reg_bn_s, p0_reg_bn_b,
         p0_cls_pred_w, p0_cls_pred_b, p0_reg_pred_w, p0_reg_pred_b),
        (p1_cls_fc_w, p1_cls_fc_b, p1_cls_conv_w, p1_cls_bn_s, p1_cls_bn_b,
         p1_reg_fc_w, p1_reg_fc_b, p1_reg_conv_w, p1_reg_bn_s, p1_reg_bn_b,
         p1_cls_pred_w, p1_cls_pred_b, p1_reg_pred_w, p1_reg_pred_b),
        (p2_cls_fc_w, p2_cls_fc_b, p2_cls_conv_w, p2_cls_bn_s, p2_cls_bn_b,
         p2_reg_fc_w, p2_reg_fc_b, p2_reg_conv_w, p2_reg_bn_s, p2_reg_bn_b,
         p2_cls_pred_w, p2_cls_pred_b, p2_reg_pred_w, p2_reg_pred_b),
    )
    cls_list, reg_list = [], []
    for feat, ps, (h, w) in zip(feats, params, _HWS):
        c, r = _head_scale(feat, ps, h, w)
        cls_list.append(c)
        reg_list.append(r)
    cls_score = jnp.concatenate(cls_list, -1)
    reg_dist = jnp.concatenate(reg_list, -1)
    anchor_points, stride_tensor = _anchors()
    return cls_score, reg_dist, anchor_points, stride_tensor


# speedup vs baseline: 1.3705x; 1.3705x over previous
import jax
import jax.numpy as jnp
from jax.experimental import pallas as pl
from jax.experimental.pallas import tpu as pltpu

_REG_MAX = 16
_NC = 80
_NREG = 4 * (_REG_MAX + 1)
_STRIDES = (32, 16, 8)
_HWS = ((20, 20), (40, 40), (80, 80))
_OFFSET = 0.5


def _head_body(H, W, NB, feat_ref,
               cfw, cfb, ccw, cbs, cbb,
               rfw, rfb, rcw, rbs, rbb,
               cpw, cpb, rpw, rpb,
               cls_out, reg_out):
    C = feat_ref.shape[1]
    L = H * W
    bf = jnp.bfloat16
    cx = jax.lax.broadcasted_iota(jnp.int32, (1, L), 1) % W
    proj = jax.lax.broadcasted_iota(
        jnp.int32, (_REG_MAX + 1, 1), 0).astype(jnp.float32)

    def conv3x3(xb, w_ref):
        zrow = jnp.zeros((C, W), bf)
        slices = (
            jnp.concatenate([zrow, xb[:, :L - W]], axis=1),
            xb,
            jnp.concatenate([xb[:, W:], zrow], axis=1),
        )
        P = []
        for kx in range(3):
            acc = None
            for ky in range(3):
                t = jnp.dot(w_ref[ky * 3 + kx], slices[ky],
                            preferred_element_type=jnp.float32)
                acc = t if acc is None else acc + t
            P.append(acc)
        left = pltpu.roll(P[0], 1, axis=1)
        right = pltpu.roll(P[2], L - 1, axis=1)
        return (P[1] + jnp.where(cx >= 1, left, 0.0)
                + jnp.where(cx <= W - 2, right, 0.0))

    for i in range(NB):
        feat = feat_ref[i]
        feat_bf = feat.astype(bf)
        avg = jnp.sum(feat, axis=1, keepdims=True) * (1.0 / L)
        avg_b = avg.astype(bf)

        def ese(fw, fb, cw, s, b):
            g = jnp.dot(fw[...], avg_b, preferred_element_type=jnp.float32)
            gate = jax.nn.sigmoid(g + fb[...]).astype(bf)
            gated = feat_bf * gate
            x = jnp.dot(cw[...], gated, preferred_element_type=jnp.float32)
            z = x * s[...] + b[...]
            return z * jax.nn.sigmoid(z)

        ese_c = ese(cfw, cfb, ccw, cbs, cbb)
        ese_r = ese(rfw, rfb, rcw, rbs, rbb)

        logit = conv3x3(ese_c.astype(bf) + feat_bf, cpw) + cpb[...]
        cls_out[i] = jax.nn.sigmoid(logit)

        r = conv3x3(ese_r.astype(bf), rpw) + rpb[...]
        rows = []
        for f in range(4):
            blk = r[17 * f:17 * (f + 1), :]
            m = jnp.max(blk, axis=0, keepdims=True)
            e = jnp.exp(blk - m)
            s = jnp.sum(e, axis=0, keepdims=True)
            wsum = jnp.sum(e * proj, axis=0, keepdims=True)
            rows.append(wsum / s)
        reg_out[i] = jnp.concatenate(rows, axis=0)


def _head_scale(feat, ps, H, W, NB=1, interpret=False):
    (cfw, cfb, ccw, cbs, cbb, rfw, rfb, rcw, rbs, rbb,
     pcw, pcb, prw, prb) = ps
    B, C, _, _ = feat.shape
    L = H * W
    bf = jnp.bfloat16
    featr = feat.reshape(B, C, L)
    col = lambda v: v.reshape(-1, 1)
    w11 = lambda w: w.reshape(w.shape[0], w.shape[1]).astype(bf)
    w33 = lambda w: w.transpose(2, 3, 0, 1).reshape(9, w.shape[0], w.shape[1]).astype(bf)

    args = (featr,
            w11(cfw), col(cfb), w11(ccw), col(cbs), col(cbb),
            w11(rfw), col(rfb), w11(rcw), col(rbs), col(rbb),
            w33(pcw), col(pcb), w33(prw), col(prb))

    full = lambda a: pl.BlockSpec(a.shape, lambda b, _n=a.ndim: (0,) * _n)
    in_specs = [pl.BlockSpec((NB, C, L), lambda b: (b, 0, 0))]
    in_specs += [full(a) for a in args[1:]]

    import functools
    body = functools.partial(_head_body, H, W, NB)
    cls_s, reg_d = pl.pallas_call(
        body,
        grid=(B // NB,),
        in_specs=in_specs,
        out_specs=[pl.BlockSpec((NB, _NC, L), lambda b: (b, 0, 0)),
                   pl.BlockSpec((NB, 4, L), lambda b: (b, 0, 0))],
        out_shape=[jax.ShapeDtypeStruct((B, _NC, L), jnp.float32),
                   jax.ShapeDtypeStruct((B, 4, L), jnp.float32)],
        compiler_params=pltpu.CompilerParams(
            dimension_semantics=("arbitrary",)),
        interpret=interpret,
    )(*args)
    return cls_s, reg_d


def _anchors():
    pts, st = [], []
    for (h, w), s in zip(_HWS, _STRIDES):
        sx = jnp.arange(w, dtype=jnp.float32) + _OFFSET
        sy = jnp.arange(h, dtype=jnp.float32) + _OFFSET
        yy, xx = jnp.meshgrid(sy, sx, indexing='ij')
        pts.append(jnp.stack([xx, yy], -1).reshape(-1, 2))
        st.append(jnp.full((h * w, 1), s, dtype=jnp.float32))
    return jnp.concatenate(pts, 0), jnp.concatenate(st, 0)


def kernel(feat0, feat1, feat2,
           p0_cls_fc_w, p0_cls_fc_b, p0_cls_conv_w, p0_cls_bn_s, p0_cls_bn_b,
           p0_reg_fc_w, p0_reg_fc_b, p0_reg_conv_w, p0_reg_bn_s, p0_reg_bn_b,
           p0_cls_pred_w, p0_cls_pred_b, p0_reg_pred_w, p0_reg_pred_b,
           p1_cls_fc_w, p1_cls_fc_b, p1_cls_conv_w, p1_cls_bn_s, p1_cls_bn_b,
           p1_reg_fc_w, p1_reg_fc_b, p1_reg_conv_w, p1_reg_bn_s, p1_reg_bn_b,
           p1_cls_pred_w, p1_cls_pred_b, p1_reg_pred_w, p1_reg_pred_b,
           p2_cls_fc_w, p2_cls_fc_b, p2_cls_conv_w, p2_cls_bn_s, p2_cls_bn_b,
           p2_reg_fc_w, p2_reg_fc_b, p2_reg_conv_w, p2_reg_bn_s, p2_reg_bn_b,
           p2_cls_pred_w, p2_cls_pred_b, p2_reg_pred_w, p2_reg_pred_b):
    feats = (feat0, feat1, feat2)
    params = (
        (p0_cls_fc_w, p0_cls_fc_b, p0_cls_conv_w, p0_cls_bn_s, p0_cls_bn_b,
         p0_reg_fc_w, p0_reg_fc_b, p0_reg_conv_w, p0_reg_bn_s, p0_reg_bn_b,
         p0_cls_pred_w, p0_cls_pred_b, p0_reg_pred_w, p0_reg_pred_b),
        (p1_cls_fc_w, p1_cls_fc_b, p1_cls_conv_w, p1_cls_bn_s, p1_cls_bn_b,
         p1_reg_fc_w, p1_reg_fc_b, p1_reg_conv_w, p1_reg_bn_s, p1_reg_bn_b,
         p1_cls_pred_w, p1_cls_pred_b, p1_reg_pred_w, p1_reg_pred_b),
        (p2_cls_fc_w, p2_cls_fc_b, p2_cls_conv_w, p2_cls_bn_s, p2_cls_bn_b,
         p2_reg_fc_w, p2_reg_fc_b, p2_reg_conv_w, p2_reg_bn_s, p2_reg_bn_b,
         p2_cls_pred_w, p2_cls_pred_b, p2_reg_pred_w, p2_reg_pred_b),
    )
    cls_list, reg_list = [], []
    for feat, ps, (h, w), nb in zip(feats, params, _HWS, (2, 2, 1)):
        c, r = _head_scale(feat, ps, h, w, NB=nb)
        cls_list.append(c)
        reg_list.append(r)
    cls_score = jnp.concatenate(cls_list, -1)
    reg_dist = jnp.concatenate(reg_list, -1)
    anchor_points, stride_tensor = _anchors()
    return cls_score, reg_dist, anchor_points, stride_tensor
